# Initial kernel scaffold; baseline (speedup 1.0000x reference)
import jax
import jax.numpy as jnp
from jax.experimental import pallas as pl


def kernel(x, W_router, w1, w2, w3):
    raise NotImplementedError("write your pallas kernel here")


# speedup vs baseline: 1.9639x; 1.9639x over previous
import jax
import jax.numpy as jnp
from jax.experimental import pallas as pl
from jax.experimental.pallas import tpu as pltpu

_NE = 8
_K = 2
_D = 1024
_F = 2048
_BM = 512
_BF = 512
_RB = 512

_INTERPRET = False


def _router_body(x_ref, wr_ref, e_ref, w_ref):
    lt = jax.lax.dot_general(
        wr_ref[...], x_ref[...], (((1,), (1,)), ((), ())),
        preferred_element_type=jnp.float32)
    rows = jax.lax.broadcasted_iota(jnp.int32, lt.shape, 0)
    v1 = jnp.max(lt, axis=0)
    a1 = jnp.min(jnp.where(lt == v1[None, :], rows, _NE), axis=0)
    lt2 = jnp.where(rows == a1[None, :], -jnp.inf, lt)
    v2 = jnp.max(lt2, axis=0)
    a2 = jnp.min(jnp.where(lt2 == v2[None, :], rows, _NE), axis=0)
    p1 = 1.0 / (1.0 + jnp.exp(v2 - v1))
    e_ref[...] = jnp.concatenate([a1[None, :], a2[None, :]], axis=0)
    w_ref[...] = jnp.concatenate([p1[None, :], (1.0 - p1)[None, :]], axis=0)


def _route(x_flat, W_router):
    n = x_flat.shape[0]
    return pl.pallas_call(
        _router_body,
        grid=(n // _RB,),
        in_specs=[
            pl.BlockSpec((_RB, _D), lambda i: (i, 0)),
            pl.BlockSpec((_NE, _D), lambda i: (0, 0)),
        ],
        out_specs=[
            pl.BlockSpec((_K, _RB), lambda i: (0, i)),
            pl.BlockSpec((_K, _RB), lambda i: (0, i)),
        ],
        out_shape=[
            jax.ShapeDtypeStruct((_K, n), jnp.int32),
            jax.ShapeDtypeStruct((_K, n), jnp.float32),
        ],
        interpret=_INTERPRET,
    )(x_flat, W_router)


def _ffn_body(bid_ref, eid_ref, lo_ref, hi_ref,
              x_ref, w1_ref, w2_ref, w3_ref, o_ref, acc_ref):
    t = pl.program_id(0)
    f = pl.program_id(1)
    nf = pl.num_programs(1)
    lo = lo_ref[t]
    hi = hi_ref[t]

    @pl.when(hi > lo)
    def _():
        xb = x_ref[...]
        g = jnp.dot(xb, w1_ref[0], preferred_element_type=jnp.float32)
        v = jnp.dot(xb, w2_ref[0], preferred_element_type=jnp.float32)
        h = (g * (1.0 / (1.0 + jnp.exp(-g)))) * v
        p = jnp.dot(h, w3_ref[0], preferred_element_type=jnp.float32)

        @pl.when(f == 0)
        def _():
            acc_ref[...] = p

        @pl.when(f > 0)
        def _():
            acc_ref[...] = acc_ref[...] + p

        @pl.when(f == nf - 1)
        def _():
            r = jax.lax.broadcasted_iota(jnp.int32, (_BM, _D), 0)
            m = (r >= lo) & (r < hi)
            o_ref[...] = jnp.where(m, acc_ref[...], o_ref[...])


def _grouped_ffn(sorted_inputs, w1, w2, w3, bid, eid, lo, hi):
    s = sorted_inputs.shape[0]
    nb = s // _BM
    t_tiles = nb + _NE
    nf = _F // _BF
    grid_spec = pltpu.PrefetchScalarGridSpec(
        num_scalar_prefetch=4,
        grid=(t_tiles, nf),
        in_specs=[
            pl.BlockSpec((_BM, _D), lambda t, f, b, e, l, h: (b[t], 0)),
            pl.BlockSpec((1, _D, _BF), lambda t, f, b, e, l, h: (e[t], 0, f)),
            pl.BlockSpec((1, _D, _BF), lambda t, f, b, e, l, h: (e[t], 0, f)),
            pl.BlockSpec((1, _BF, _D), lambda t, f, b, e, l, h: (e[t], f, 0)),
        ],
        out_specs=pl.BlockSpec((_BM, _D), lambda t, f, b, e, l, h: (b[t], 0)),
        scratch_shapes=[pltpu.VMEM((_BM, _D), jnp.float32)],
    )
    return pl.pallas_call(
        _ffn_body,
        grid_spec=grid_spec,
        out_shape=jax.ShapeDtypeStruct((s, _D), jnp.float32),
        compiler_params=pltpu.CompilerParams(
            dimension_semantics=("arbitrary", "arbitrary")),
        interpret=_INTERPRET,
    )(bid, eid, lo, hi, sorted_inputs, w1, w2, w3)


def kernel(x, W_router, w1, w2, w3):
    batch, seq, d = x.shape
    x_flat = x.reshape(-1, d)
    n = x_flat.shape[0]
    s = n * _K

    e2, p2 = _route(x_flat, W_router)
    flat_e = e2.T.reshape(-1)
    flat_w = p2.T.reshape(-1)

    order = jnp.argsort(flat_e, stable=True)
    sorted_tok = (order // _K).astype(jnp.int32)
    inv = jnp.zeros((s,), jnp.int32).at[order].set(
        jnp.arange(s, dtype=jnp.int32))
    counts = jnp.sum(
        (flat_e[None, :] == jnp.arange(_NE, dtype=flat_e.dtype)[:, None])
        .astype(jnp.int32), axis=1)
    offsets = jnp.concatenate(
        [jnp.zeros((1,), jnp.int32), jnp.cumsum(counts, dtype=jnp.int32)])
    nb = s // _BM
    starts = jnp.sort(jnp.concatenate(
        [jnp.arange(nb, dtype=jnp.int32) * _BM, offsets[:_NE]]))
    ends = jnp.concatenate([starts[1:], jnp.array([s], jnp.int32)])
    bid = jnp.minimum(starts // _BM, nb - 1)
    eid = jnp.minimum(
        jnp.searchsorted(offsets[1:], starts, side='right'),
        _NE - 1).astype(jnp.int32)
    lo = starts - bid * _BM
    hi = ends - bid * _BM

    sorted_inputs = jnp.take(x_flat, sorted_tok, axis=0)
    sorted_out = _grouped_ffn(sorted_inputs, w1, w2, w3, bid, eid, lo, hi)

    per = jnp.take(sorted_out, inv, axis=0) * flat_w[:, None]
    out_flat = per[0::2] + per[1::2]
    return out_flat.reshape(batch, seq, d)
